# baseline (speedup 1.0000x reference)
import functools
import math

import jax
import jax.numpy as jnp
from jax.experimental import pallas as pl
from jax.experimental.pallas import tpu as pltpu

TOPK = 2
EPS = 1e-5


def _rms_norm(x):
    v = jnp.mean(jnp.square(x), axis=-1, keepdims=True)
    return x * jax.lax.rsqrt(v + EPS)


def _routing_kernel(h0_ref, i0_ref, gw_ref, idx_ref, w_ref, aux_ref, *, B, E):
    hs0 = h0_ref[...] + i0_ref[...]
    logits = jnp.dot(hs0, gw_ref[...], preferred_element_type=jnp.float32)
    m = jnp.max(logits, axis=-1, keepdims=True)
    e = jnp.exp(logits - m)
    p = e / jnp.sum(e, axis=-1, keepdims=True)
    ids = jax.lax.broadcasted_iota(jnp.int32, (B, E), 1)
    m1 = jnp.max(p, axis=-1, keepdims=True)
    i1 = jnp.min(jnp.where(p == m1, ids, E), axis=-1, keepdims=True)
    pm = jnp.where(ids == i1, -1.0, p)
    m2 = jnp.max(pm, axis=-1, keepdims=True)
    i2 = jnp.min(jnp.where(pm == m2, ids, E), axis=-1, keepdims=True)
    s12 = jnp.maximum(m1 + m2, 1e-8)
    importance = jnp.sum(p, axis=0, keepdims=True) / B
    sel = ((ids == i1) | (ids == i2)).astype(jnp.float32)
    load = jnp.sum(sel, axis=0, keepdims=True) / (B * TOPK)
    aux = jnp.sum(E * importance * load)
    idx_ref[...] = jnp.concatenate([i1, i2], axis=1)
    w_ref[...] = jnp.concatenate([m1 / s12, m2 / s12], axis=1)
    aux_ref[...] = aux[None, None]


def _hs_kernel(hid_ref, inj_ref, hs_ref):
    hs_ref[...] = hid_ref[...] + inj_ref[...]


def _down_kernel(idx_ref, hs_ref, dW_ref, h_ref):
    h_ref[0] = jnp.dot(hs_ref[0], dW_ref[0], preferred_element_type=jnp.float32)


def _qkv_kernel(idx_ref, h_ref, cos_ref, sin_ref, qkvW_ref, qkv_ref,
                *, NH_SUB, HD):
    qkv = jnp.dot(h_ref[0], qkvW_ref[0], preferred_element_type=jnp.float32)
    c = cos_ref[...]
    s = sin_ref[...]
    half = HD // 2

    def rope(v):
        rot = jnp.concatenate([-v[:, half:], v[:, :half]], axis=1)
        return v * c + rot * s

    HSUB = NH_SUB * HD
    pieces = [rope(qkv[:, j * HD:(j + 1) * HD]) for j in range(2 * NH_SUB)]
    pieces.append(qkv[:, 2 * HSUB:])
    qkv_ref[0] = jnp.concatenate(pieces, axis=1)


def _attn_kernel(q_ref, k_ref, v_ref, o_ref, *, HD, HEADS_PER_STEP):
    scale = 1.0 / math.sqrt(HD)
    pieces = []
    for sub in range(HEADS_PER_STEP):
        q = q_ref[0][:, sub * HD:(sub + 1) * HD] * scale
        k = k_ref[0][:, sub * HD:(sub + 1) * HD]
        v = v_ref[0][:, sub * HD:(sub + 1) * HD]
        s = jax.lax.dot_general(q, k, (((1,), (1,)), ((), ())),
                                preferred_element_type=jnp.float32)
        m = jnp.max(s, axis=-1, keepdims=True)
        p = jnp.exp(s - m)
        r = jnp.sum(p, axis=-1, keepdims=True)
        o = jnp.dot(p, v, preferred_element_type=jnp.float32)
        pieces.append(o / r)
    o_ref[0] = jnp.concatenate(pieces, axis=1)


def _onorm_kernel(idx_ref, h_ref, attn_ref, oW_ref, h2_ref):
    t = jnp.dot(attn_ref[0], oW_ref[0], preferred_element_type=jnp.float32)
    h2_ref[0] = _rms_norm(h_ref[0] + t)


def _gu_kernel(idx_ref, h2_ref, guW_ref, act_ref, *, INTER):
    gu = jnp.dot(h2_ref[0], guW_ref[0], preferred_element_type=jnp.float32)
    g = gu[:, :INTER]
    u = gu[:, INTER:]
    act_ref[0] = g * jax.lax.logistic(g) * u


def _dp_kernel(idx_ref, h2_ref, act_ref, dpW_ref, h3_ref):
    t = jnp.dot(act_ref[0], dpW_ref[0], preferred_element_type=jnp.float32)
    h3_ref[0] = _rms_norm(h2_ref[0] + t)


def _up_mix_kernel(idx_ref, w_ref, h3_ref, upW_ref, hs_ref, out_ref):
    row = pl.program_id(0)
    k = pl.program_id(2)
    a = row * TOPK + k
    w = w_ref[a]
    y = jnp.dot(h3_ref[0], upW_ref[0], preferred_element_type=jnp.float32) * w

    @pl.when(k == 0)
    def _():
        out_ref[0] = y

    @pl.when(k == TOPK - 1)
    def _():
        x = hs_ref[0] + out_ref[0] + y
        out_ref[0] = _rms_norm(x)


def kernel(hidden_states, cos, sin, input_injection, gate_W, down_W, qkv_W,
           o_W, gu_W, dp_W, up_W):
    B, S, H = hidden_states.shape
    HD = cos.shape[-1]
    E = gate_W.shape[-1]
    HSUB = down_W.shape[-1]
    NH_SUB = HSUB // HD
    INTER = dp_W.shape[1]
    A = B * TOPK

    f32 = jnp.float32

    St0 = min(512, S)
    hs = pl.pallas_call(
        _hs_kernel,
        grid=(B, S // St0),
        in_specs=[
            pl.BlockSpec((1, St0, H), lambda b, s: (b, s, 0)),
            pl.BlockSpec((1, St0, H), lambda b, s: (b, s, 0)),
        ],
        out_specs=pl.BlockSpec((1, St0, H), lambda b, s: (b, s, 0)),
        out_shape=jax.ShapeDtypeStruct((B, S, H), f32),
    )(hidden_states, input_injection)

    idx2, w2, aux = pl.pallas_call(
        functools.partial(_routing_kernel, B=B, E=E),
        out_shape=(
            jax.ShapeDtypeStruct((B, TOPK), jnp.int32),
            jax.ShapeDtypeStruct((B, TOPK), f32),
            jax.ShapeDtypeStruct((1, 1), f32),
        ),
    )(hidden_states[:, 0], input_injection[:, 0], gate_W)
    idx_flat = idx2.reshape(A)
    w_flat = w2.reshape(A)

    St = min(512, S)
    h = pl.pallas_call(
        _down_kernel,
        grid_spec=pltpu.PrefetchScalarGridSpec(
            num_scalar_prefetch=1,
            grid=(A, S // St),
            in_specs=[
                pl.BlockSpec((1, St, H), lambda a, s, idx: (a // TOPK, s, 0)),
                pl.BlockSpec((1, H, HSUB), lambda a, s, idx: (idx[a], 0, 0)),
            ],
            out_specs=pl.BlockSpec((1, St, HSUB), lambda a, s, idx: (a, s, 0)),
        ),
        out_shape=jax.ShapeDtypeStruct((A, S, HSUB), f32),
    )(idx_flat, hs, down_W)

    qkv = pl.pallas_call(
        functools.partial(_qkv_kernel, NH_SUB=NH_SUB, HD=HD),
        grid_spec=pltpu.PrefetchScalarGridSpec(
            num_scalar_prefetch=1,
            grid=(A, S // St),
            in_specs=[
                pl.BlockSpec((1, St, HSUB), lambda a, s, idx: (a, s, 0)),
                pl.BlockSpec((St, HD), lambda a, s, idx: (s, 0)),
                pl.BlockSpec((St, HD), lambda a, s, idx: (s, 0)),
                pl.BlockSpec((1, HSUB, 3 * HSUB),
                             lambda a, s, idx: (idx[a], 0, 0)),
            ],
            out_specs=pl.BlockSpec((1, St, 3 * HSUB),
                                   lambda a, s, idx: (a, s, 0)),
        ),
        out_shape=jax.ShapeDtypeStruct((A, S, 3 * HSUB), f32),
    )(idx_flat, h, cos, sin, qkv_W)

    Sq = min(512, S)
    HPS = 2 if NH_SUB % 2 == 0 else 1
    HB = HPS * HD
    attn = pl.pallas_call(
        functools.partial(_attn_kernel, HD=HD, HEADS_PER_STEP=HPS),
        grid=(A, NH_SUB // HPS, S // Sq),
        in_specs=[
            pl.BlockSpec((1, Sq, HB), lambda a, hh, sq: (a, sq, hh)),
            pl.BlockSpec((1, S, HB),
                         lambda a, hh, sq: (a, 0, NH_SUB // HPS + hh)),
            pl.BlockSpec((1, S, HB),
                         lambda a, hh, sq: (a, 0, 2 * (NH_SUB // HPS) + hh)),
        ],
        out_specs=pl.BlockSpec((1, Sq, HB), lambda a, hh, sq: (a, sq, hh)),
        out_shape=jax.ShapeDtypeStruct((A, S, HSUB), f32),
    )(qkv, qkv, qkv)

    h2 = pl.pallas_call(
        _onorm_kernel,
        grid_spec=pltpu.PrefetchScalarGridSpec(
            num_scalar_prefetch=1,
            grid=(A, S // St),
            in_specs=[
                pl.BlockSpec((1, St, HSUB), lambda a, s, idx: (a, s, 0)),
                pl.BlockSpec((1, St, HSUB), lambda a, s, idx: (a, s, 0)),
                pl.BlockSpec((1, HSUB, HSUB), lambda a, s, idx: (idx[a], 0, 0)),
            ],
            out_specs=pl.BlockSpec((1, St, HSUB), lambda a, s, idx: (a, s, 0)),
        ),
        out_shape=jax.ShapeDtypeStruct((A, S, HSUB), f32),
    )(idx_flat, h, attn, o_W)

    Se = min(256, S)
    act = pl.pallas_call(
        functools.partial(_gu_kernel, INTER=INTER),
        grid_spec=pltpu.PrefetchScalarGridSpec(
            num_scalar_prefetch=1,
            grid=(A, S // Se),
            in_specs=[
                pl.BlockSpec((1, Se, HSUB), lambda a, s, idx: (a, s, 0)),
                pl.BlockSpec((1, HSUB, 2 * INTER),
                             lambda a, s, idx: (idx[a], 0, 0)),
            ],
            out_specs=pl.BlockSpec((1, Se, INTER), lambda a, s, idx: (a, s, 0)),
        ),
        out_shape=jax.ShapeDtypeStruct((A, S, INTER), f32),
    )(idx_flat, h2, gu_W)

    h3 = pl.pallas_call(
        _dp_kernel,
        grid_spec=pltpu.PrefetchScalarGridSpec(
            num_scalar_prefetch=1,
            grid=(A, S // St),
            in_specs=[
                pl.BlockSpec((1, St, HSUB), lambda a, s, idx: (a, s, 0)),
                pl.BlockSpec((1, St, INTER), lambda a, s, idx: (a, s, 0)),
                pl.BlockSpec((1, INTER, HSUB), lambda a, s, idx: (idx[a], 0, 0)),
            ],
            out_specs=pl.BlockSpec((1, St, HSUB), lambda a, s, idx: (a, s, 0)),
        ),
        out_shape=jax.ShapeDtypeStruct((A, S, HSUB), f32),
    )(idx_flat, h2, act, dp_W)

    out = pl.pallas_call(
        _up_mix_kernel,
        grid_spec=pltpu.PrefetchScalarGridSpec(
            num_scalar_prefetch=1,
            grid=(B, S // St, TOPK),
            in_specs=[
                pl.BlockSpec(memory_space=pltpu.SMEM),
                pl.BlockSpec((1, St, HSUB),
                             lambda r, s, k, idx: (r * TOPK + k, s, 0)),
                pl.BlockSpec((1, HSUB, H),
                             lambda r, s, k, idx: (idx[r * TOPK + k], 0, 0)),
                pl.BlockSpec((1, St, H), lambda r, s, k, idx: (r, s, 0)),
            ],
            out_specs=pl.BlockSpec((1, St, H), lambda r, s, k, idx: (r, s, 0)),
        ),
        out_shape=jax.ShapeDtypeStruct((B, S, H), f32),
    )(idx_flat, w_flat, h3, up_W, hs)

    return out, aux.reshape(())


# speedup vs baseline: 1.0683x; 1.0683x over previous
import functools
import math

import jax
import jax.numpy as jnp
from jax.experimental import pallas as pl
from jax.experimental.pallas import tpu as pltpu

TOPK = 2
EPS = 1e-5


def _rms_norm(x):
    v = jnp.mean(jnp.square(x), axis=-1, keepdims=True)
    return x * jax.lax.rsqrt(v + EPS)


def _routing_kernel(h0_ref, i0_ref, gw_ref, idx_ref, w_ref, aux_ref, *, B, E):
    hs0 = h0_ref[...] + i0_ref[...]
    logits = jnp.dot(hs0, gw_ref[...], preferred_element_type=jnp.float32)
    m = jnp.max(logits, axis=-1, keepdims=True)
    e = jnp.exp(logits - m)
    p = e / jnp.sum(e, axis=-1, keepdims=True)
    ids = jax.lax.broadcasted_iota(jnp.int32, (B, E), 1)
    m1 = jnp.max(p, axis=-1, keepdims=True)
    i1 = jnp.min(jnp.where(p == m1, ids, E), axis=-1, keepdims=True)
    pm = jnp.where(ids == i1, -1.0, p)
    m2 = jnp.max(pm, axis=-1, keepdims=True)
    i2 = jnp.min(jnp.where(pm == m2, ids, E), axis=-1, keepdims=True)
    s12 = jnp.maximum(m1 + m2, 1e-8)
    importance = jnp.sum(p, axis=0, keepdims=True) / B
    sel = ((ids == i1) | (ids == i2)).astype(jnp.float32)
    load = jnp.sum(sel, axis=0, keepdims=True) / (B * TOPK)
    aux = jnp.sum(E * importance * load)
    idx_ref[...] = jnp.concatenate([i1, i2], axis=1)
    w_ref[...] = jnp.concatenate([m1 / s12, m2 / s12], axis=1)
    aux_ref[...] = aux[None, None]


def _hs_kernel(hid_ref, inj_ref, hs_ref):
    hs_ref[...] = hid_ref[...] + inj_ref[...]


def _down_kernel(idx_ref, hs_ref, dW_ref, h_ref):
    h_ref[0] = jnp.dot(hs_ref[0], dW_ref[0], preferred_element_type=jnp.float32)


def _qkv_kernel(idx_ref, h_ref, cos_ref, sin_ref, qkvW_ref, qkv_ref,
                *, NH_SUB, HD):
    qkv = jnp.dot(h_ref[0], qkvW_ref[0], preferred_element_type=jnp.float32)
    c = cos_ref[...]
    s = sin_ref[...]
    half = HD // 2

    def rope(v):
        rot = jnp.concatenate([-v[:, half:], v[:, :half]], axis=1)
        return v * c + rot * s

    HSUB = NH_SUB * HD
    pieces = [rope(qkv[:, j * HD:(j + 1) * HD]) for j in range(2 * NH_SUB)]
    pieces.append(qkv[:, 2 * HSUB:])
    qkv_ref[0] = jnp.concatenate(pieces, axis=1)


def _attn_kernel(q_ref, k_ref, v_ref, o_ref, *, HD, HEADS_PER_STEP):
    scale = 1.0 / math.sqrt(HD)
    pieces = []
    for sub in range(HEADS_PER_STEP):
        q = q_ref[0][:, sub * HD:(sub + 1) * HD] * scale
        k = k_ref[0][:, sub * HD:(sub + 1) * HD]
        v = v_ref[0][:, sub * HD:(sub + 1) * HD]
        s = jax.lax.dot_general(q, k, (((1,), (1,)), ((), ())),
                                preferred_element_type=jnp.float32)
        m = jnp.max(s, axis=-1, keepdims=True)
        p = jnp.exp(s - m)
        r = jnp.sum(p, axis=-1, keepdims=True)
        o = jnp.dot(p, v, preferred_element_type=jnp.float32)
        pieces.append(o / r)
    o_ref[0] = jnp.concatenate(pieces, axis=1)


def _onorm_kernel(idx_ref, h_ref, attn_ref, oW_ref, h2_ref):
    t = jnp.dot(attn_ref[0], oW_ref[0], preferred_element_type=jnp.float32)
    h2_ref[0] = _rms_norm(h_ref[0] + t)


def _gu_kernel(idx_ref, h2_ref, guW_ref, act_ref, *, INTER):
    gu = jnp.dot(h2_ref[0], guW_ref[0], preferred_element_type=jnp.float32)
    g = gu[:, :INTER]
    u = gu[:, INTER:]
    act_ref[0] = g * jax.lax.logistic(g) * u


def _dp_kernel(idx_ref, h2_ref, act_ref, dpW_ref, h3_ref):
    t = jnp.dot(act_ref[0], dpW_ref[0], preferred_element_type=jnp.float32)
    h3_ref[0] = _rms_norm(h2_ref[0] + t)


def _up_mix_kernel(idx_ref, w_ref, h3_ref, upW_ref, hs_ref, out_ref):
    row = pl.program_id(0)
    k = pl.program_id(2)
    a = row * TOPK + k
    w = w_ref[a]
    y = jnp.dot(h3_ref[0], upW_ref[0], preferred_element_type=jnp.float32) * w

    @pl.when(k == 0)
    def _():
        out_ref[0] = y

    @pl.when(k == TOPK - 1)
    def _():
        x = hs_ref[0] + out_ref[0] + y
        out_ref[0] = _rms_norm(x)


def kernel(hidden_states, cos, sin, input_injection, gate_W, down_W, qkv_W,
           o_W, gu_W, dp_W, up_W):
    B, S, H = hidden_states.shape
    HD = cos.shape[-1]
    E = gate_W.shape[-1]
    HSUB = down_W.shape[-1]
    NH_SUB = HSUB // HD
    INTER = dp_W.shape[1]
    A = B * TOPK

    f32 = jnp.float32

    St0 = min(512, S)
    hs = pl.pallas_call(
        _hs_kernel,
        grid=(B, S // St0),
        in_specs=[
            pl.BlockSpec((1, St0, H), lambda b, s: (b, s, 0)),
            pl.BlockSpec((1, St0, H), lambda b, s: (b, s, 0)),
        ],
        out_specs=pl.BlockSpec((1, St0, H), lambda b, s: (b, s, 0)),
        out_shape=jax.ShapeDtypeStruct((B, S, H), f32),
    )(hidden_states, input_injection)

    idx2, w2, aux = pl.pallas_call(
        functools.partial(_routing_kernel, B=B, E=E),
        out_shape=(
            jax.ShapeDtypeStruct((B, TOPK), jnp.int32),
            jax.ShapeDtypeStruct((B, TOPK), f32),
            jax.ShapeDtypeStruct((1, 1), f32),
        ),
    )(hidden_states[:, 0], input_injection[:, 0], gate_W)
    idx_flat = idx2.reshape(A)
    w_flat = w2.reshape(A)

    St = min(512, S)
    h = pl.pallas_call(
        _down_kernel,
        grid_spec=pltpu.PrefetchScalarGridSpec(
            num_scalar_prefetch=1,
            grid=(A, S // St),
            in_specs=[
                pl.BlockSpec((1, St, H), lambda a, s, idx: (a // TOPK, s, 0)),
                pl.BlockSpec((1, H, HSUB), lambda a, s, idx: (idx[a], 0, 0)),
            ],
            out_specs=pl.BlockSpec((1, St, HSUB), lambda a, s, idx: (a, s, 0)),
        ),
        out_shape=jax.ShapeDtypeStruct((A, S, HSUB), f32),
    )(idx_flat, hs, down_W)

    qkv = pl.pallas_call(
        functools.partial(_qkv_kernel, NH_SUB=NH_SUB, HD=HD),
        grid_spec=pltpu.PrefetchScalarGridSpec(
            num_scalar_prefetch=1,
            grid=(A, S // St),
            in_specs=[
                pl.BlockSpec((1, St, HSUB), lambda a, s, idx: (a, s, 0)),
                pl.BlockSpec((St, HD), lambda a, s, idx: (s, 0)),
                pl.BlockSpec((St, HD), lambda a, s, idx: (s, 0)),
                pl.BlockSpec((1, HSUB, 3 * HSUB),
                             lambda a, s, idx: (idx[a], 0, 0)),
            ],
            out_specs=pl.BlockSpec((1, St, 3 * HSUB),
                                   lambda a, s, idx: (a, s, 0)),
        ),
        out_shape=jax.ShapeDtypeStruct((A, S, 3 * HSUB), f32),
    )(idx_flat, h, cos, sin, qkv_W)

    Sq = min(512, S)
    HPS = 4 if NH_SUB % 4 == 0 else (2 if NH_SUB % 2 == 0 else 1)
    HB = HPS * HD
    attn = pl.pallas_call(
        functools.partial(_attn_kernel, HD=HD, HEADS_PER_STEP=HPS),
        grid=(A, NH_SUB // HPS, S // Sq),
        in_specs=[
            pl.BlockSpec((1, Sq, HB), lambda a, hh, sq: (a, sq, hh)),
            pl.BlockSpec((1, S, HB),
                         lambda a, hh, sq: (a, 0, NH_SUB // HPS + hh)),
            pl.BlockSpec((1, S, HB),
                         lambda a, hh, sq: (a, 0, 2 * (NH_SUB // HPS) + hh)),
        ],
        out_specs=pl.BlockSpec((1, Sq, HB), lambda a, hh, sq: (a, sq, hh)),
        out_shape=jax.ShapeDtypeStruct((A, S, HSUB), f32),
    )(qkv, qkv, qkv)

    h2 = pl.pallas_call(
        _onorm_kernel,
        grid_spec=pltpu.PrefetchScalarGridSpec(
            num_scalar_prefetch=1,
            grid=(A, S // St),
            in_specs=[
                pl.BlockSpec((1, St, HSUB), lambda a, s, idx: (a, s, 0)),
                pl.BlockSpec((1, St, HSUB), lambda a, s, idx: (a, s, 0)),
                pl.BlockSpec((1, HSUB, HSUB), lambda a, s, idx: (idx[a], 0, 0)),
            ],
            out_specs=pl.BlockSpec((1, St, HSUB), lambda a, s, idx: (a, s, 0)),
        ),
        out_shape=jax.ShapeDtypeStruct((A, S, HSUB), f32),
    )(idx_flat, h, attn, o_W)

    Se = min(256, S)
    act = pl.pallas_call(
        functools.partial(_gu_kernel, INTER=INTER),
        grid_spec=pltpu.PrefetchScalarGridSpec(
            num_scalar_prefetch=1,
            grid=(A, S // Se),
            in_specs=[
                pl.BlockSpec((1, Se, HSUB), lambda a, s, idx: (a, s, 0)),
                pl.BlockSpec((1, HSUB, 2 * INTER),
                             lambda a, s, idx: (idx[a], 0, 0)),
            ],
            out_specs=pl.BlockSpec((1, Se, INTER), lambda a, s, idx: (a, s, 0)),
        ),
        out_shape=jax.ShapeDtypeStruct((A, S, INTER), f32),
    )(idx_flat, h2, gu_W)

    h3 = pl.pallas_call(
        _dp_kernel,
        grid_spec=pltpu.PrefetchScalarGridSpec(
            num_scalar_prefetch=1,
            grid=(A, S // St),
            in_specs=[
                pl.BlockSpec((1, St, HSUB), lambda a, s, idx: (a, s, 0)),
                pl.BlockSpec((1, St, INTER), lambda a, s, idx: (a, s, 0)),
                pl.BlockSpec((1, INTER, HSUB), lambda a, s, idx: (idx[a], 0, 0)),
            ],
            out_specs=pl.BlockSpec((1, St, HSUB), lambda a, s, idx: (a, s, 0)),
        ),
        out_shape=jax.ShapeDtypeStruct((A, S, HSUB), f32),
    )(idx_flat, h2, act, dp_W)

    out = pl.pallas_call(
        _up_mix_kernel,
        grid_spec=pltpu.PrefetchScalarGridSpec(
            num_scalar_prefetch=1,
            grid=(B, S // St, TOPK),
            in_specs=[
                pl.BlockSpec(memory_space=pltpu.SMEM),
                pl.BlockSpec((1, St, HSUB),
                             lambda r, s, k, idx: (r * TOPK + k, s, 0)),
                pl.BlockSpec((1, HSUB, H),
                             lambda r, s, k, idx: (idx[r * TOPK + k], 0, 0)),
                pl.BlockSpec((1, St, H), lambda r, s, k, idx: (r, s, 0)),
            ],
            out_specs=pl.BlockSpec((1, St, H), lambda r, s, k, idx: (r, s, 0)),
        ),
        out_shape=jax.ShapeDtypeStruct((B, S, H), f32),
    )(idx_flat, w_flat, h3, up_W, hs)

    return out, aux.reshape(())


# speedup vs baseline: 1.0857x; 1.0164x over previous
import functools
import math

import jax
import jax.numpy as jnp
from jax.experimental import pallas as pl
from jax.experimental.pallas import tpu as pltpu

TOPK = 2
EPS = 1e-5


def _rms_norm(x):
    v = jnp.mean(jnp.square(x), axis=-1, keepdims=True)
    return x * jax.lax.rsqrt(v + EPS)


def _routing_kernel(h0_ref, i0_ref, gw_ref, idx_ref, w_ref, aux_ref, *, B, E):
    hs0 = h0_ref[...] + i0_ref[...]
    logits = jnp.dot(hs0, gw_ref[...], preferred_element_type=jnp.float32)
    m = jnp.max(logits, axis=-1, keepdims=True)
    e = jnp.exp(logits - m)
    p = e / jnp.sum(e, axis=-1, keepdims=True)
    ids = jax.lax.broadcasted_iota(jnp.int32, (B, E), 1)
    m1 = jnp.max(p, axis=-1, keepdims=True)
    i1 = jnp.min(jnp.where(p == m1, ids, E), axis=-1, keepdims=True)
    pm = jnp.where(ids == i1, -1.0, p)
    m2 = jnp.max(pm, axis=-1, keepdims=True)
    i2 = jnp.min(jnp.where(pm == m2, ids, E), axis=-1, keepdims=True)
    s12 = jnp.maximum(m1 + m2, 1e-8)
    importance = jnp.sum(p, axis=0, keepdims=True) / B
    sel = ((ids == i1) | (ids == i2)).astype(jnp.float32)
    load = jnp.sum(sel, axis=0, keepdims=True) / (B * TOPK)
    aux = jnp.sum(E * importance * load)
    idx_ref[...] = jnp.concatenate([i1, i2], axis=1)
    w_ref[...] = jnp.concatenate([m1 / s12, m2 / s12], axis=1)
    aux_ref[...] = aux[None, None]


def _hs_kernel(hid_ref, inj_ref, hs_ref):
    hs_ref[...] = hid_ref[...] + inj_ref[...]


def _down_kernel(idx_ref, hs_ref, dW_ref, h_ref):
    h_ref[0] = jnp.dot(hs_ref[0], dW_ref[0], preferred_element_type=jnp.float32)


def _qkv_kernel(idx_ref, h_ref, cos_ref, sin_ref, qkvW_ref, qkv_ref,
                *, NH_SUB, HD):
    qkv = jnp.dot(h_ref[0], qkvW_ref[0], preferred_element_type=jnp.float32)
    c = cos_ref[...]
    s = sin_ref[...]
    half = HD // 2

    def rope(v):
        rot = jnp.concatenate([-v[:, half:], v[:, :half]], axis=1)
        return v * c + rot * s

    HSUB = NH_SUB * HD
    pieces = [rope(qkv[:, j * HD:(j + 1) * HD]) for j in range(2 * NH_SUB)]
    pieces.append(qkv[:, 2 * HSUB:])
    qkv_ref[0] = jnp.concatenate(pieces, axis=1)


def _attn_kernel(q_ref, k_ref, v_ref, o_ref, *, HD, HEADS_PER_STEP):
    scale = 1.0 / math.sqrt(HD)
    pieces = []
    for sub in range(HEADS_PER_STEP):
        q = q_ref[0][:, sub * HD:(sub + 1) * HD] * scale
        k = k_ref[0][:, sub * HD:(sub + 1) * HD]
        v = v_ref[0][:, sub * HD:(sub + 1) * HD]
        s = jax.lax.dot_general(q, k, (((1,), (1,)), ((), ())),
                                preferred_element_type=jnp.float32)
        m = jnp.max(s, axis=-1, keepdims=True)
        p = jnp.exp(s - m)
        r = jnp.sum(p, axis=-1, keepdims=True)
        o = jnp.dot(p, v, preferred_element_type=jnp.float32)
        pieces.append(o / r)
    o_ref[0] = jnp.concatenate(pieces, axis=1)


def _onorm_kernel(idx_ref, h_ref, attn_ref, oW_ref, h2_ref):
    t = jnp.dot(attn_ref[0], oW_ref[0], preferred_element_type=jnp.float32)
    h2_ref[0] = _rms_norm(h_ref[0] + t)


def _gu_kernel(idx_ref, h2_ref, guW_ref, act_ref, *, INTER):
    gu = jnp.dot(h2_ref[0], guW_ref[0], preferred_element_type=jnp.float32)
    g = gu[:, :INTER]
    u = gu[:, INTER:]
    act_ref[0] = g * jax.lax.logistic(g) * u


def _dp_kernel(idx_ref, h2_ref, act_ref, dpW_ref, h3_ref):
    t = jnp.dot(act_ref[0], dpW_ref[0], preferred_element_type=jnp.float32)
    h3_ref[0] = _rms_norm(h2_ref[0] + t)


def _up_mix_kernel(idx_ref, w_ref, h3_ref, upW_ref, hs_ref, out_ref):
    row = pl.program_id(0)
    k = pl.program_id(2)
    a = row * TOPK + k
    w = w_ref[a]
    y = jnp.dot(h3_ref[0], upW_ref[0], preferred_element_type=jnp.float32) * w

    @pl.when(k == 0)
    def _():
        out_ref[0] = y

    @pl.when(k == TOPK - 1)
    def _():
        x = hs_ref[0] + out_ref[0] + y
        out_ref[0] = _rms_norm(x)


def kernel(hidden_states, cos, sin, input_injection, gate_W, down_W, qkv_W,
           o_W, gu_W, dp_W, up_W):
    B, S, H = hidden_states.shape
    HD = cos.shape[-1]
    E = gate_W.shape[-1]
    HSUB = down_W.shape[-1]
    NH_SUB = HSUB // HD
    INTER = dp_W.shape[1]
    A = B * TOPK

    f32 = jnp.float32

    St0 = min(512, S)
    hs = pl.pallas_call(
        _hs_kernel,
        grid=(B, S // St0),
        in_specs=[
            pl.BlockSpec((1, St0, H), lambda b, s: (b, s, 0)),
            pl.BlockSpec((1, St0, H), lambda b, s: (b, s, 0)),
        ],
        out_specs=pl.BlockSpec((1, St0, H), lambda b, s: (b, s, 0)),
        out_shape=jax.ShapeDtypeStruct((B, S, H), f32),
    )(hidden_states, input_injection)

    idx2, w2, aux = pl.pallas_call(
        functools.partial(_routing_kernel, B=B, E=E),
        out_shape=(
            jax.ShapeDtypeStruct((B, TOPK), jnp.int32),
            jax.ShapeDtypeStruct((B, TOPK), f32),
            jax.ShapeDtypeStruct((1, 1), f32),
        ),
    )(hidden_states[:, 0], input_injection[:, 0], gate_W)
    idx_flat = idx2.reshape(A)
    w_flat = w2.reshape(A)

    St = min(512, S)
    h = pl.pallas_call(
        _down_kernel,
        grid_spec=pltpu.PrefetchScalarGridSpec(
            num_scalar_prefetch=1,
            grid=(A, S // St),
            in_specs=[
                pl.BlockSpec((1, St, H), lambda a, s, idx: (a // TOPK, s, 0)),
                pl.BlockSpec((1, H, HSUB), lambda a, s, idx: (idx[a], 0, 0)),
            ],
            out_specs=pl.BlockSpec((1, St, HSUB), lambda a, s, idx: (a, s, 0)),
        ),
        out_shape=jax.ShapeDtypeStruct((A, S, HSUB), f32),
    )(idx_flat, hs, down_W)

    qkv = pl.pallas_call(
        functools.partial(_qkv_kernel, NH_SUB=NH_SUB, HD=HD),
        grid_spec=pltpu.PrefetchScalarGridSpec(
            num_scalar_prefetch=1,
            grid=(A, S // St),
            in_specs=[
                pl.BlockSpec((1, St, HSUB), lambda a, s, idx: (a, s, 0)),
                pl.BlockSpec((St, HD), lambda a, s, idx: (s, 0)),
                pl.BlockSpec((St, HD), lambda a, s, idx: (s, 0)),
                pl.BlockSpec((1, HSUB, 3 * HSUB),
                             lambda a, s, idx: (idx[a], 0, 0)),
            ],
            out_specs=pl.BlockSpec((1, St, 3 * HSUB),
                                   lambda a, s, idx: (a, s, 0)),
        ),
        out_shape=jax.ShapeDtypeStruct((A, S, 3 * HSUB), f32),
    )(idx_flat, h, cos, sin, qkv_W)

    HPS = NH_SUB
    Sq = min(256, S)
    HB = HPS * HD
    attn = pl.pallas_call(
        functools.partial(_attn_kernel, HD=HD, HEADS_PER_STEP=HPS),
        grid=(A, NH_SUB // HPS, S // Sq),
        in_specs=[
            pl.BlockSpec((1, Sq, HB), lambda a, hh, sq: (a, sq, hh)),
            pl.BlockSpec((1, S, HB),
                         lambda a, hh, sq: (a, 0, NH_SUB // HPS + hh)),
            pl.BlockSpec((1, S, HB),
                         lambda a, hh, sq: (a, 0, 2 * (NH_SUB // HPS) + hh)),
        ],
        out_specs=pl.BlockSpec((1, Sq, HB), lambda a, hh, sq: (a, sq, hh)),
        out_shape=jax.ShapeDtypeStruct((A, S, HSUB), f32),
    )(qkv, qkv, qkv)

    h2 = pl.pallas_call(
        _onorm_kernel,
        grid_spec=pltpu.PrefetchScalarGridSpec(
            num_scalar_prefetch=1,
            grid=(A, S // St),
            in_specs=[
                pl.BlockSpec((1, St, HSUB), lambda a, s, idx: (a, s, 0)),
                pl.BlockSpec((1, St, HSUB), lambda a, s, idx: (a, s, 0)),
                pl.BlockSpec((1, HSUB, HSUB), lambda a, s, idx: (idx[a], 0, 0)),
            ],
            out_specs=pl.BlockSpec((1, St, HSUB), lambda a, s, idx: (a, s, 0)),
        ),
        out_shape=jax.ShapeDtypeStruct((A, S, HSUB), f32),
    )(idx_flat, h, attn, o_W)

    Se = min(256, S)
    act = pl.pallas_call(
        functools.partial(_gu_kernel, INTER=INTER),
        grid_spec=pltpu.PrefetchScalarGridSpec(
            num_scalar_prefetch=1,
            grid=(A, S // Se),
            in_specs=[
                pl.BlockSpec((1, Se, HSUB), lambda a, s, idx: (a, s, 0)),
                pl.BlockSpec((1, HSUB, 2 * INTER),
                             lambda a, s, idx: (idx[a], 0, 0)),
            ],
            out_specs=pl.BlockSpec((1, Se, INTER), lambda a, s, idx: (a, s, 0)),
        ),
        out_shape=jax.ShapeDtypeStruct((A, S, INTER), f32),
    )(idx_flat, h2, gu_W)

    h3 = pl.pallas_call(
        _dp_kernel,
        grid_spec=pltpu.PrefetchScalarGridSpec(
            num_scalar_prefetch=1,
            grid=(A, S // St),
            in_specs=[
                pl.BlockSpec((1, St, HSUB), lambda a, s, idx: (a, s, 0)),
                pl.BlockSpec((1, St, INTER), lambda a, s, idx: (a, s, 0)),
                pl.BlockSpec((1, INTER, HSUB), lambda a, s, idx: (idx[a], 0, 0)),
            ],
            out_specs=pl.BlockSpec((1, St, HSUB), lambda a, s, idx: (a, s, 0)),
        ),
        out_shape=jax.ShapeDtypeStruct((A, S, HSUB), f32),
    )(idx_flat, h2, act, dp_W)

    out = pl.pallas_call(
        _up_mix_kernel,
        grid_spec=pltpu.PrefetchScalarGridSpec(
            num_scalar_prefetch=1,
            grid=(B, S // St, TOPK),
            in_specs=[
                pl.BlockSpec(memory_space=pltpu.SMEM),
                pl.BlockSpec((1, St, HSUB),
                             lambda r, s, k, idx: (r * TOPK + k, s, 0)),
                pl.BlockSpec((1, HSUB, H),
                             lambda r, s, k, idx: (idx[r * TOPK + k], 0, 0)),
                pl.BlockSpec((1, St, H), lambda r, s, k, idx: (r, s, 0)),
            ],
            out_specs=pl.BlockSpec((1, St, H), lambda r, s, k, idx: (r, s, 0)),
        ),
        out_shape=jax.ShapeDtypeStruct((B, S, H), f32),
    )(idx_flat, w_flat, h3, up_W, hs)

    return out, aux.reshape(())


# speedup vs baseline: 1.1298x; 1.0406x over previous
import functools
import math

import jax
import jax.numpy as jnp
from jax.experimental import pallas as pl
from jax.experimental.pallas import tpu as pltpu

TOPK = 2
EPS = 1e-5


def _rms_norm(x):
    v = jnp.mean(jnp.square(x), axis=-1, keepdims=True)
    return x * jax.lax.rsqrt(v + EPS)


def _routing_kernel(h0_ref, i0_ref, gw_ref, idx_ref, w_ref, aux_ref, *, B, E):
    hs0 = h0_ref[...] + i0_ref[...]
    logits = jnp.dot(hs0, gw_ref[...], preferred_element_type=jnp.float32)
    m = jnp.max(logits, axis=-1, keepdims=True)
    e = jnp.exp(logits - m)
    p = e / jnp.sum(e, axis=-1, keepdims=True)
    ids = jax.lax.broadcasted_iota(jnp.int32, (B, E), 1)
    m1 = jnp.max(p, axis=-1, keepdims=True)
    i1 = jnp.min(jnp.where(p == m1, ids, E), axis=-1, keepdims=True)
    pm = jnp.where(ids == i1, -1.0, p)
    m2 = jnp.max(pm, axis=-1, keepdims=True)
    i2 = jnp.min(jnp.where(pm == m2, ids, E), axis=-1, keepdims=True)
    s12 = jnp.maximum(m1 + m2, 1e-8)
    importance = jnp.sum(p, axis=0, keepdims=True) / B
    sel = ((ids == i1) | (ids == i2)).astype(jnp.float32)
    load = jnp.sum(sel, axis=0, keepdims=True) / (B * TOPK)
    aux = jnp.sum(E * importance * load)
    idx_ref[...] = jnp.concatenate([i1, i2], axis=1)
    w_ref[...] = jnp.concatenate([m1 / s12, m2 / s12], axis=1)
    aux_ref[...] = aux[None, None]


def _hs_kernel(hid_ref, inj_ref, hs_ref):
    hs_ref[...] = hid_ref[...] + inj_ref[...]


def _down_qkv_kernel(idx_ref, hs_ref, cos_ref, sin_ref, dW_ref, qkvW_ref,
                     h_ref, qkv_ref, *, NH_SUB, HD):
    h = jnp.dot(hs_ref[0], dW_ref[0], preferred_element_type=jnp.float32)
    h_ref[0] = h
    qkv = jnp.dot(h, qkvW_ref[0], preferred_element_type=jnp.float32)
    c = cos_ref[...]
    s = sin_ref[...]
    half = HD // 2

    def rope(v):
        rot = jnp.concatenate([-v[:, half:], v[:, :half]], axis=1)
        return v * c + rot * s

    HSUB = NH_SUB * HD
    pieces = [rope(qkv[:, j * HD:(j + 1) * HD]) for j in range(2 * NH_SUB)]
    pieces.append(qkv[:, 2 * HSUB:])
    qkv_ref[0] = jnp.concatenate(pieces, axis=1)


def _attn_onorm_kernel(idx_ref, q_ref, k_ref, v_ref, h_ref, oW_ref, h2_ref,
                       *, HD, NH_SUB):
    scale = 1.0 / math.sqrt(HD)
    pieces = []
    for sub in range(NH_SUB):
        q = q_ref[0][:, sub * HD:(sub + 1) * HD] * scale
        k = k_ref[0][:, sub * HD:(sub + 1) * HD]
        v = v_ref[0][:, sub * HD:(sub + 1) * HD]
        s = jax.lax.dot_general(q, k, (((1,), (1,)), ((), ())),
                                preferred_element_type=jnp.float32)
        m = jnp.max(s, axis=-1, keepdims=True)
        p = jnp.exp(s - m)
        r = jnp.sum(p, axis=-1, keepdims=True)
        o = jnp.dot(p, v, preferred_element_type=jnp.float32)
        pieces.append(o / r)
    attn = jnp.concatenate(pieces, axis=1)
    t = jnp.dot(attn, oW_ref[0], preferred_element_type=jnp.float32)
    h2_ref[0] = _rms_norm(h_ref[0] + t)


def _gu_kernel(idx_ref, h2_ref, guW_ref, act_ref, *, INTER):
    gu = jnp.dot(h2_ref[0], guW_ref[0], preferred_element_type=jnp.float32)
    g = gu[:, :INTER]
    u = gu[:, INTER:]
    act_ref[0] = g * jax.lax.logistic(g) * u


def _dp_kernel(idx_ref, h2_ref, act_ref, dpW_ref, h3_ref):
    t = jnp.dot(act_ref[0], dpW_ref[0], preferred_element_type=jnp.float32)
    h3_ref[0] = _rms_norm(h2_ref[0] + t)


def _up_mix_kernel(idx_ref, w_ref, h3_ref, upW_ref, hs_ref, out_ref):
    row = pl.program_id(0)
    k = pl.program_id(2)
    a = row * TOPK + k
    w = w_ref[a]
    y = jnp.dot(h3_ref[0], upW_ref[0], preferred_element_type=jnp.float32) * w

    @pl.when(k == 0)
    def _():
        out_ref[0] = y

    @pl.when(k == TOPK - 1)
    def _():
        x = hs_ref[0] + out_ref[0] + y
        out_ref[0] = _rms_norm(x)


def kernel(hidden_states, cos, sin, input_injection, gate_W, down_W, qkv_W,
           o_W, gu_W, dp_W, up_W):
    B, S, H = hidden_states.shape
    HD = cos.shape[-1]
    E = gate_W.shape[-1]
    HSUB = down_W.shape[-1]
    NH_SUB = HSUB // HD
    INTER = dp_W.shape[1]
    A = B * TOPK

    f32 = jnp.float32

    St0 = min(512, S)
    hs = pl.pallas_call(
        _hs_kernel,
        grid=(B, S // St0),
        in_specs=[
            pl.BlockSpec((1, St0, H), lambda b, s: (b, s, 0)),
            pl.BlockSpec((1, St0, H), lambda b, s: (b, s, 0)),
        ],
        out_specs=pl.BlockSpec((1, St0, H), lambda b, s: (b, s, 0)),
        out_shape=jax.ShapeDtypeStruct((B, S, H), f32),
    )(hidden_states, input_injection)

    idx2, w2, aux = pl.pallas_call(
        functools.partial(_routing_kernel, B=B, E=E),
        out_shape=(
            jax.ShapeDtypeStruct((B, TOPK), jnp.int32),
            jax.ShapeDtypeStruct((B, TOPK), f32),
            jax.ShapeDtypeStruct((1, 1), f32),
        ),
    )(hidden_states[:, 0], input_injection[:, 0], gate_W)
    idx_flat = idx2.reshape(A)
    w_flat = w2.reshape(A)

    St = min(512, S)
    Sf = min(256, S)
    h, qkv = pl.pallas_call(
        functools.partial(_down_qkv_kernel, NH_SUB=NH_SUB, HD=HD),
        grid_spec=pltpu.PrefetchScalarGridSpec(
            num_scalar_prefetch=1,
            grid=(A, S // Sf),
            in_specs=[
                pl.BlockSpec((1, Sf, H), lambda a, s, idx: (a // TOPK, s, 0)),
                pl.BlockSpec((Sf, HD), lambda a, s, idx: (s, 0)),
                pl.BlockSpec((Sf, HD), lambda a, s, idx: (s, 0)),
                pl.BlockSpec((1, H, HSUB), lambda a, s, idx: (idx[a], 0, 0)),
                pl.BlockSpec((1, HSUB, 3 * HSUB),
                             lambda a, s, idx: (idx[a], 0, 0)),
            ],
            out_specs=[
                pl.BlockSpec((1, Sf, HSUB), lambda a, s, idx: (a, s, 0)),
                pl.BlockSpec((1, Sf, 3 * HSUB), lambda a, s, idx: (a, s, 0)),
            ],
        ),
        out_shape=(
            jax.ShapeDtypeStruct((A, S, HSUB), f32),
            jax.ShapeDtypeStruct((A, S, 3 * HSUB), f32),
        ),
    )(idx_flat, hs, cos, sin, down_W, qkv_W)

    Sq = min(256, S)
    h2 = pl.pallas_call(
        functools.partial(_attn_onorm_kernel, HD=HD, NH_SUB=NH_SUB),
        grid_spec=pltpu.PrefetchScalarGridSpec(
            num_scalar_prefetch=1,
            grid=(A, S // Sq),
            in_specs=[
                pl.BlockSpec((1, Sq, HSUB), lambda a, s, idx: (a, s, 0)),
                pl.BlockSpec((1, S, HSUB), lambda a, s, idx: (a, 0, 1)),
                pl.BlockSpec((1, S, HSUB), lambda a, s, idx: (a, 0, 2)),
                pl.BlockSpec((1, Sq, HSUB), lambda a, s, idx: (a, s, 0)),
                pl.BlockSpec((1, HSUB, HSUB), lambda a, s, idx: (idx[a], 0, 0)),
            ],
            out_specs=pl.BlockSpec((1, Sq, HSUB), lambda a, s, idx: (a, s, 0)),
        ),
        out_shape=jax.ShapeDtypeStruct((A, S, HSUB), f32),
    )(idx_flat, qkv, qkv, qkv, h, o_W)

    Se = min(256, S)
    act = pl.pallas_call(
        functools.partial(_gu_kernel, INTER=INTER),
        grid_spec=pltpu.PrefetchScalarGridSpec(
            num_scalar_prefetch=1,
            grid=(A, S // Se),
            in_specs=[
                pl.BlockSpec((1, Se, HSUB), lambda a, s, idx: (a, s, 0)),
                pl.BlockSpec((1, HSUB, 2 * INTER),
                             lambda a, s, idx: (idx[a], 0, 0)),
            ],
            out_specs=pl.BlockSpec((1, Se, INTER), lambda a, s, idx: (a, s, 0)),
        ),
        out_shape=jax.ShapeDtypeStruct((A, S, INTER), f32),
    )(idx_flat, h2, gu_W)

    h3 = pl.pallas_call(
        _dp_kernel,
        grid_spec=pltpu.PrefetchScalarGridSpec(
            num_scalar_prefetch=1,
            grid=(A, S // St),
            in_specs=[
                pl.BlockSpec((1, St, HSUB), lambda a, s, idx: (a, s, 0)),
                pl.BlockSpec((1, St, INTER), lambda a, s, idx: (a, s, 0)),
                pl.BlockSpec((1, INTER, HSUB), lambda a, s, idx: (idx[a], 0, 0)),
            ],
            out_specs=pl.BlockSpec((1, St, HSUB), lambda a, s, idx: (a, s, 0)),
        ),
        out_shape=jax.ShapeDtypeStruct((A, S, HSUB), f32),
    )(idx_flat, h2, act, dp_W)

    out = pl.pallas_call(
        _up_mix_kernel,
        grid_spec=pltpu.PrefetchScalarGridSpec(
            num_scalar_prefetch=1,
            grid=(B, S // St, TOPK),
            in_specs=[
                pl.BlockSpec(memory_space=pltpu.SMEM),
                pl.BlockSpec((1, St, HSUB),
                             lambda r, s, k, idx: (r * TOPK + k, s, 0)),
                pl.BlockSpec((1, HSUB, H),
                             lambda r, s, k, idx: (idx[r * TOPK + k], 0, 0)),
                pl.BlockSpec((1, St, H), lambda r, s, k, idx: (r, s, 0)),
            ],
            out_specs=pl.BlockSpec((1, St, H), lambda r, s, k, idx: (r, s, 0)),
        ),
        out_shape=jax.ShapeDtypeStruct((B, S, H), f32),
    )(idx_flat, w_flat, h3, up_W, hs)

    return out, aux.reshape(())
